# baseline (speedup 1.0000x reference)
import jax
import jax.numpy as jnp
from jax import lax
from jax.experimental import pallas as pl
from jax.experimental.pallas import tpu as pltpu
from jax.experimental.pallas import tpu_sc as plsc

B = 16384
F = 100
FH = F // 2
D = 128
L = 16
NC, NS = 2, 16
NW = NC * NS
BPW = B // NW
ND = D // L
BPAD = 128
NBUF = 4


def _fm_body(x_hbm, emb_hbm, bias_hbm, off_hbm, out_hbm,
             idx_v, rows0, rows1, rows2, rows3, bias0, bias1, bias2, bias3,
             part_v, out_v, off_v, sem0, sem1, sem2, sem3):
    rows_bufs = (rows0, rows1, rows2, rows3)
    bias_bufs = (bias0, bias1, bias2, bias3)
    sems = (sem0, sem1, sem2, sem3)

    wid = lax.axis_index("s") * NC + lax.axis_index("c")
    base = wid * 2 * BPW

    pltpu.sync_copy(x_hbm.at[pl.ds(base, 2 * BPW)], idx_v)
    pltpu.sync_copy(off_hbm, off_v)
    off_vec = off_v[...]

    for k in range(NBUF):
        bias_bufs[k][pl.ds(48, L)] = jnp.zeros((L,), jnp.float32)
        bias_bufs[k][pl.ds(BPAD - L, L)] = jnp.zeros((L,), jnp.float32)

    def issue(b, k):
        pltpu.async_copy(emb_hbm.at[idx_v.at[2 * b]],
                         rows_bufs[k].at[pl.ds(0, FH)], sems[k])
        pltpu.async_copy(emb_hbm.at[idx_v.at[2 * b + 1]],
                         rows_bufs[k].at[pl.ds(FH, FH)], sems[k])
        pltpu.async_copy(bias_hbm.at[idx_v.at[2 * b]],
                         bias_bufs[k].at[pl.ds(0, FH)], sems[k])
        pltpu.async_copy(bias_hbm.at[idx_v.at[2 * b + 1]],
                         bias_bufs[k].at[pl.ds(64, FH)], sems[k])

    def wait(b, k):
        pltpu.make_async_copy(emb_hbm.at[idx_v.at[2 * b]],
                              rows_bufs[k].at[pl.ds(0, FH)], sems[k]).wait()
        pltpu.make_async_copy(emb_hbm.at[idx_v.at[2 * b + 1]],
                              rows_bufs[k].at[pl.ds(FH, FH)], sems[k]).wait()
        pltpu.make_async_copy(bias_hbm.at[idx_v.at[2 * b]],
                              bias_bufs[k].at[pl.ds(0, FH)], sems[k]).wait()
        pltpu.make_async_copy(bias_hbm.at[idx_v.at[2 * b + 1]],
                              bias_bufs[k].at[pl.ds(64, FH)], sems[k]).wait()

    def compute_row(b, k):
        rows, bias = rows_bufs[k], bias_bufs[k]

        def fbody(f, accs):
            out = []
            for d in range(ND):
                v = rows[f, pl.ds(d * L, L)]
                out.append(accs[d] + v)
            for d in range(ND):
                v = rows[f, pl.ds(d * L, L)]
                out.append(accs[ND + d] + v * v)
            return tuple(out)

        init = (jnp.zeros((L,), jnp.float32),) * (2 * ND)
        accs = lax.fori_loop(0, F, fbody, init, unroll=2)
        fm = accs[0] * accs[0] - accs[ND]
        for d in range(1, ND):
            fm = fm + (accs[d] * accs[d] - accs[ND + d])
        bsum = bias[pl.ds(0, L)]
        for j in range(1, BPAD // L):
            bsum = bsum + bias[pl.ds(j * L, L)]
        part_v[b, :] = fm * 0.5 + bsum

    for k in range(NBUF):
        issue(k, k)

    def quad_body(i, _):
        for k in range(NBUF):
            b = NBUF * i + k
            wait(b, k)
            compute_row(b, k)

            @pl.when(b + NBUF < BPW)
            def _():
                issue(b + NBUF, k)
        return 0

    lax.fori_loop(0, BPW // NBUF, quad_body, 0)

    lane = lax.iota(jnp.int32, L)

    def g_body(g, _):
        ridx = g * L + lane
        s = jnp.zeros((L,), jnp.float32)
        for c in range(L):
            cidx = jnp.full((L,), c, jnp.int32)
            s = s + plsc.load_gather(part_v, [ridx, cidx])
        s = s + off_vec
        y = 5.0 / (1.0 + jnp.exp(-s)) + 0.5
        out_v[pl.ds(g * L, L)] = y
        return 0

    lax.fori_loop(0, BPW // L, g_body, 0)
    pltpu.sync_copy(out_v, out_hbm.at[pl.ds(wid * BPW, BPW)])


_fm_call = pl.kernel(
    _fm_body,
    out_type=jax.ShapeDtypeStruct((B,), jnp.float32),
    mesh=plsc.VectorSubcoreMesh(core_axis_name="c", subcore_axis_name="s",
                                num_cores=NC, num_subcores=NS),
    compiler_params=pltpu.CompilerParams(needs_layout_passes=False,
                                         use_tc_tiling_on_sc=False),
    scratch_types=[
        pltpu.VMEM((2 * BPW, FH), jnp.int32),
        pltpu.VMEM((F, D), jnp.float32),
        pltpu.VMEM((F, D), jnp.float32),
        pltpu.VMEM((F, D), jnp.float32),
        pltpu.VMEM((F, D), jnp.float32),
        pltpu.VMEM((BPAD,), jnp.float32),
        pltpu.VMEM((BPAD,), jnp.float32),
        pltpu.VMEM((BPAD,), jnp.float32),
        pltpu.VMEM((BPAD,), jnp.float32),
        pltpu.VMEM((BPW, L), jnp.float32),
        pltpu.VMEM((BPW,), jnp.float32),
        pltpu.VMEM((L,), jnp.float32),
        pltpu.SemaphoreType.DMA,
        pltpu.SemaphoreType.DMA,
        pltpu.SemaphoreType.DMA,
        pltpu.SemaphoreType.DMA,
    ],
)


def kernel(X, x_emb_weight, x_bias, offset):
    off16 = jnp.broadcast_to(offset.astype(jnp.float32), (L,))
    x2 = X.astype(jnp.int32).reshape(2 * B, FH)
    return _fm_call(x2, x_emb_weight, x_bias, off16)
